# baseline (speedup 1.0000x reference)
import functools

import jax
import jax.numpy as jnp
from jax.experimental import pallas as pl
from jax.experimental.pallas import tpu as pltpu

EMA_EPS = 1e-12


def _bmm(a, b):
    return jax.lax.dot_general(
        a, b,
        dimension_numbers=(((2,), (1,)), ((0,), (0,))),
        preferred_element_type=jnp.float32,
    )


def _ema_block_kernel(pt_ref, z_ref, out_ref, state_ref, *, T):
    k = pl.program_id(0)
    B, _, D = z_ref.shape

    p = pt_ref[:, 0, :]
    logd = jnp.log(jnp.maximum(1.0 - p, EMA_EPS))

    jj = jax.lax.broadcasted_iota(jnp.int32, (T, T), 0)
    ii = jax.lax.broadcasted_iota(jnp.int32, (T, T), 1)
    cum_mat = jnp.where(jj <= ii, 1.0, 0.0)
    S = jnp.dot(logd, cum_mat, preferred_element_type=jnp.float32)

    Sp = S - jnp.log(p)
    delta = S[:, :, None] - Sp[:, None, :]
    delta = jnp.where((jj >= ii)[None], delta, -jnp.inf)
    W = jnp.exp(delta)

    acc = _bmm(W, z_ref[...])

    @pl.when(k == 0)
    def _():
        state_ref[...] = jnp.zeros((B, 8, D), jnp.float32)

    lane8 = jax.lax.broadcasted_iota(jnp.int32, (T, 8), 1)
    cw = jnp.where(lane8 == 0, jnp.exp(S)[:, :, None], 0.0)
    res = acc + _bmm(cw, state_ref[...])

    out_ref[...] = res
    state_ref[:, 0, :] = res[:, T - 1, :]


@jax.jit
def kernel(z, pt):
    B, L, D = z.shape
    T = 256
    K = L // T

    body = functools.partial(_ema_block_kernel, T=T)
    return pl.pallas_call(
        body,
        grid=(K,),
        in_specs=[
            pl.BlockSpec((B, 1, T), lambda k: (0, 0, k)),
            pl.BlockSpec((B, T, D), lambda k: (0, k, 0)),
        ],
        out_specs=pl.BlockSpec((B, T, D), lambda k: (0, k, 0)),
        out_shape=jax.ShapeDtypeStruct((B, L, D), jnp.float32),
        scratch_shapes=[pltpu.VMEM((B, 8, D), jnp.float32)],
    )(pt.reshape(B, 1, L), z)


# speedup vs baseline: 1.1252x; 1.1252x over previous
import functools

import jax
import jax.numpy as jnp
from jax.experimental import pallas as pl
from jax.experimental.pallas import tpu as pltpu

EMA_EPS = 1e-12


def _bmm(a, b):
    return jax.lax.dot_general(
        a, b,
        dimension_numbers=(((2,), (1,)), ((0,), (0,))),
        preferred_element_type=jnp.float32,
    )


def _ema_block_kernel(pt_ref, z_ref, out_ref, state_ref, *, T):
    k = pl.program_id(0)
    B, _, D = z_ref.shape

    p = pt_ref[:, 0, :]
    logd = jnp.log(jnp.maximum(1.0 - p, EMA_EPS))

    jj = jax.lax.broadcasted_iota(jnp.int32, (T, T), 0)
    ii = jax.lax.broadcasted_iota(jnp.int32, (T, T), 1)
    cum_mat = jnp.where(jj <= ii, 1.0, 0.0)
    S = jnp.dot(logd, cum_mat, preferred_element_type=jnp.float32)

    Sp = S - jnp.log(p)
    delta = S[:, :, None] - Sp[:, None, :]
    delta = jnp.where((jj >= ii)[None], delta, -jnp.inf)
    W = jnp.exp(delta)

    acc = _bmm(W, z_ref[...])

    @pl.when(k == 0)
    def _():
        state_ref[...] = jnp.zeros((B, D), jnp.float32)

    state = state_ref[...]
    res = acc + jnp.exp(S)[:, :, None] * state[:, None, :]

    out_ref[...] = res
    state_ref[...] = res[:, T - 1, :]


@jax.jit
def kernel(z, pt):
    B, L, D = z.shape
    T = 256
    K = L // T

    body = functools.partial(_ema_block_kernel, T=T)
    return pl.pallas_call(
        body,
        grid=(K,),
        in_specs=[
            pl.BlockSpec((B, 1, T), lambda k: (0, 0, k)),
            pl.BlockSpec((B, T, D), lambda k: (0, k, 0)),
        ],
        out_specs=pl.BlockSpec((B, T, D), lambda k: (0, k, 0)),
        out_shape=jax.ShapeDtypeStruct((B, L, D), jnp.float32),
        scratch_shapes=[pltpu.VMEM((B, D), jnp.float32)],
    )(pt.reshape(B, 1, L), z)


# speedup vs baseline: 1.2137x; 1.0787x over previous
import functools

import jax
import jax.numpy as jnp
from jax.experimental import pallas as pl
from jax.experimental.pallas import tpu as pltpu

EMA_EPS = 1e-12


def _bmm(a, b):
    return jax.lax.dot_general(
        a, b,
        dimension_numbers=(((2,), (1,)), ((0,), (0,))),
        preferred_element_type=jnp.float32,
    )


def _ema_kernel(pt_ref, z_ref, out_ref, zb_ref, ob_ref, ld_sem, st_sem, *,
                T, K):
    B = pt_ref.shape[0]

    loads = []
    for k in range(K):
        ld = pltpu.make_async_copy(
            z_ref.at[:, pl.ds(k * T, T), :], zb_ref.at[k], ld_sem.at[k])
        ld.start()
        loads.append(ld)

    jj = jax.lax.broadcasted_iota(jnp.int32, (T, T), 0)
    ii = jax.lax.broadcasted_iota(jnp.int32, (T, T), 1)
    cum_mat = jnp.where(jj <= ii, 1.0, 0.0)
    tril = (jj >= ii)[None]

    def build(p):
        logd = jnp.log(jnp.maximum(1.0 - p, EMA_EPS))
        S = jnp.dot(logd, cum_mat, preferred_element_type=jnp.float32)
        Sp = S - jnp.log(p)
        delta = S[:, :, None] - Sp[:, None, :]
        delta = jnp.where(tril, delta, -jnp.inf)
        return jnp.exp(delta), jnp.exp(S)

    Ws = [build(pt_ref[:, 0, k * T:(k + 1) * T]) for k in range(K)]

    state = None
    stores = []
    for k in range(K):
        W, cw = Ws[k]
        loads[k].wait()
        res = _bmm(W, zb_ref[k])
        if state is not None:
            res = res + cw[:, :, None] * state[:, None, :]
        ob_ref[k] = res
        st = pltpu.make_async_copy(
            ob_ref.at[k], out_ref.at[:, pl.ds(k * T, T), :], st_sem.at[k])
        st.start()
        stores.append(st)
        state = res[:, T - 1, :]

    for st in stores:
        st.wait()


@jax.jit
def kernel(z, pt):
    B, L, D = z.shape
    K = 4
    T = L // K

    body = functools.partial(_ema_kernel, T=T, K=K)
    return pl.pallas_call(
        body,
        grid=(1,),
        in_specs=[
            pl.BlockSpec((B, 1, L), lambda i: (0, 0, 0)),
            pl.BlockSpec(memory_space=pl.ANY),
        ],
        out_specs=pl.BlockSpec(memory_space=pl.ANY),
        out_shape=jax.ShapeDtypeStruct((B, L, D), jnp.float32),
        scratch_shapes=[
            pltpu.VMEM((K, B, T, D), jnp.float32),
            pltpu.VMEM((K, B, T, D), jnp.float32),
            pltpu.SemaphoreType.DMA((K,)),
            pltpu.SemaphoreType.DMA((K,)),
        ],
    )(pt.reshape(B, 1, L), z)


# speedup vs baseline: 1.2303x; 1.0136x over previous
import functools

import jax
import jax.numpy as jnp
from jax.experimental import pallas as pl
from jax.experimental.pallas import tpu as pltpu

EMA_EPS = 1e-12


def _bmm(a, b):
    return jax.lax.dot_general(
        a, b,
        dimension_numbers=(((2,), (1,)), ((0,), (0,))),
        preferred_element_type=jnp.float32,
    )


def _ema_kernel(pt_ref, z_ref, out_ref, zb_ref, ob_ref, ld_sem, st_sem, *,
                T, K):
    B = pt_ref.shape[0]

    loads = []
    for k in range(K):
        ld = pltpu.make_async_copy(
            z_ref.at[:, pl.ds(k * T, T), :], zb_ref.at[k], ld_sem.at[k])
        ld.start()
        loads.append(ld)

    jj = jax.lax.broadcasted_iota(jnp.int32, (T, T), 0)
    ii = jax.lax.broadcasted_iota(jnp.int32, (T, T), 1)
    cum_mat = jnp.where(jj <= ii, 1.0, 0.0)
    tril = (jj >= ii)[None]

    def build(p):
        logd = jnp.log(jnp.maximum(1.0 - p, EMA_EPS))
        S = jnp.dot(logd, cum_mat, preferred_element_type=jnp.float32)
        Sp = S - jnp.log(p)
        delta = S[:, :, None] - Sp[:, None, :]
        delta = jnp.where(tril, delta, -jnp.inf)
        return jnp.exp(delta), jnp.exp(S)

    Ws = [build(pt_ref[:, 0, k * T:(k + 1) * T]) for k in range(K)]

    state = None
    stores = []
    for k in range(K):
        W, cw = Ws[k]
        loads[k].wait()
        res = _bmm(W, zb_ref[k])
        if state is not None:
            res = res + cw[:, :, None] * state[:, None, :]
        ob_ref[k] = res
        st = pltpu.make_async_copy(
            ob_ref.at[k], out_ref.at[:, pl.ds(k * T, T), :], st_sem.at[k])
        st.start()
        stores.append(st)
        state = res[:, T - 1, :]

    for st in stores:
        st.wait()


@jax.jit
def kernel(z, pt):
    B, L, D = z.shape
    K = 8
    T = L // K

    body = functools.partial(_ema_kernel, T=T, K=K)
    return pl.pallas_call(
        body,
        grid=(1,),
        in_specs=[
            pl.BlockSpec((B, 1, L), lambda i: (0, 0, 0)),
            pl.BlockSpec(memory_space=pl.ANY),
        ],
        out_specs=pl.BlockSpec(memory_space=pl.ANY),
        out_shape=jax.ShapeDtypeStruct((B, L, D), jnp.float32),
        scratch_shapes=[
            pltpu.VMEM((K, B, T, D), jnp.float32),
            pltpu.VMEM((K, B, T, D), jnp.float32),
            pltpu.SemaphoreType.DMA((K,)),
            pltpu.SemaphoreType.DMA((K,)),
        ],
    )(pt.reshape(B, 1, L), z)
